# Initial kernel scaffold; baseline (speedup 1.0000x reference)
import jax
import jax.numpy as jnp
from jax.experimental import pallas as pl


def kernel(mol_reprs, node_features, edge_features, edges, batch_indices, Wn, bn, We, be, Wm, bm, Wu, bu, W1, b1, W2, b2, W3, b3, W4, b4):
    raise NotImplementedError("write your pallas kernel here")


# speedup vs baseline: 2.7252x; 2.7252x over previous
import functools

import jax
import jax.numpy as jnp
from jax import lax
from jax.experimental import pallas as pl
from jax.experimental.pallas import tpu as pltpu
from jax.experimental.pallas import tpu_sc as plsc

B = 1024
N = 8192
E = 16384
MR = 256
FN = 64
FE = 16
H = 128
HE = 64
MAX_ATOMS = 24
NUM_STEPS = 8

NC = 2
NS = 16
NW = NC * NS

OUT_ROWS = B * MAX_ATOMS
OUT_PAD = OUT_ROWS + NW
HALF = OUT_ROWS // NC


def _mesh():
    return plsc.VectorSubcoreMesh(
        core_axis_name="c", subcore_axis_name="s", num_cores=NC, num_subcores=NS
    )


def _zero_vmem(ref, rows, cols):
    z = jnp.zeros((16,), jnp.float32)
    cpr = cols // 16

    def body(i, _):
        r = i // cpr
        c = (i % cpr) * 16
        ref[r, pl.ds(c, 16)] = z
        return 0

    lax.fori_loop(0, rows * cpr, body, 0)


def _sc_gather(table, idx2d, d):
    n_chunks = idx2d.shape[0]
    rows = n_chunks * 128
    cpw = n_chunks // NW
    rpw = rows // NW

    @functools.partial(
        pl.kernel,
        out_type=jax.ShapeDtypeStruct((rows, d), jnp.float32),
        mesh=_mesh(),
        scratch_types=[
            pltpu.VMEM((cpw, 128), jnp.int32),
            pltpu.VMEM((rpw, d), jnp.float32),
            pltpu.SemaphoreType.DMA,
        ],
    )
    def k(table_hbm, idx_hbm, out_hbm, idx_v, rows_v, sem):
        wid = lax.axis_index("s") * NC + lax.axis_index("c")
        pltpu.sync_copy(idx_hbm.at[pl.ds(wid * cpw, cpw)], idx_v)
        descs = [
            pltpu.async_copy(
                table_hbm.at[idx_v.at[j]], rows_v.at[pl.ds(j * 128, 128)], sem
            )
            for j in range(cpw)
        ]
        for dsc in descs:
            dsc.wait()
        pltpu.sync_copy(rows_v, out_hbm.at[pl.ds(wid * rpw, rpw)])

    return k(table, idx2d)


def _sc_segment_sum(msg, dst2d):
    epw = E // NW
    cpw = epw // 128
    npt = N // NS

    @functools.partial(
        pl.kernel,
        out_type=jax.ShapeDtypeStruct((NC, N, H), jnp.float32),
        mesh=_mesh(),
        scratch_types=[
            pltpu.VMEM((cpw, 128), jnp.int32),
            pltpu.VMEM((128, H), jnp.float32),
            pltpu.VMEM_SHARED((N, H), jnp.float32),
            pltpu.SemaphoreType.DMA,
        ],
    )
    def k(msg_hbm, dst_hbm, out_hbm, idx_v, rows_v, acc, sem):
        cid = lax.axis_index("c")
        sid = lax.axis_index("s")
        wid = sid * NC + cid
        d_idx = pltpu.async_copy(dst_hbm.at[pl.ds(wid * cpw, cpw)], idx_v, sem)
        _zero_vmem(rows_v, 128, H)
        for q in range(npt // 128):
            pltpu.sync_copy(rows_v, acc.at[pl.ds(sid * npt + q * 128, 128)])
        plsc.subcore_barrier()
        d_idx.wait()
        for j in range(cpw):
            pltpu.sync_copy(msg_hbm.at[pl.ds(wid * epw + j * 128, 128)], rows_v)
            pltpu.sync_copy(rows_v, acc.at[idx_v.at[j]], add=True)
        plsc.subcore_barrier()
        pltpu.sync_copy(
            acc.at[pl.ds(sid * npt, npt)], out_hbm.at[cid, pl.ds(sid * npt, npt)]
        )

    return k(msg, dst2d)


def _sc_final_scatter(wgt, d2):
    rpw = N // NS
    cpw = rpw // 128
    zrows = HALF // NS

    @functools.partial(
        pl.kernel,
        out_type=jax.ShapeDtypeStruct((OUT_PAD, H), jnp.float32),
        mesh=_mesh(),
        scratch_types=[
            pltpu.VMEM((cpw, 128), jnp.int32),
            pltpu.VMEM((rpw, H), jnp.float32),
            pltpu.VMEM((256, H), jnp.float32),
            pltpu.SemaphoreType.DMA,
        ],
    )
    def k(wgt_hbm, d_hbm, out_hbm, idx_v, rows_v, zbuf, sem):
        cid = lax.axis_index("c")
        sid = lax.axis_index("s")
        d_idx = pltpu.async_copy(d_hbm.at[cid, pl.ds(sid * cpw, cpw)], idx_v, sem)
        d_rows = pltpu.async_copy(wgt_hbm.at[pl.ds(sid * rpw, rpw)], rows_v, sem)
        _zero_vmem(zbuf, 256, H)
        for q in range(zrows // 256):
            pltpu.sync_copy(
                zbuf, out_hbm.at[pl.ds(cid * HALF + sid * zrows + q * 256, 256)]
            )
        plsc.subcore_barrier()
        d_idx.wait()
        d_rows.wait()
        for j in range(cpw):
            pltpu.sync_copy(rows_v.at[pl.ds(j * 128, 128)], out_hbm.at[idx_v.at[j]])

    return k(wgt, d2)


def _relu(x):
    return jnp.maximum(x, 0.0)


def _tc_init_h(nf, Wn, bn):
    def body(nf_ref, wn_ref, bn_ref, out_ref):
        out_ref[...] = _relu(nf_ref[...] @ wn_ref[...] + bn_ref[...])

    blk = 1024
    return pl.pallas_call(
        body,
        grid=(N // blk,),
        in_specs=[
            pl.BlockSpec((blk, FN), lambda i: (i, 0)),
            pl.BlockSpec((FN, H), lambda i: (0, 0)),
            pl.BlockSpec((1, H), lambda i: (0, 0)),
        ],
        out_specs=pl.BlockSpec((blk, H), lambda i: (i, 0)),
        out_shape=jax.ShapeDtypeStruct((N, H), jnp.float32),
    )(nf, Wn, bn)


def _tc_init_c(ef, We, be, Wm2, bm):
    def body(ef_ref, we_ref, be_ref, wm2_ref, bm_ref, out_ref):
        eh = _relu(ef_ref[...] @ we_ref[...] + be_ref[...])
        out_ref[...] = eh @ wm2_ref[...] + bm_ref[...]

    blk = 2048
    return pl.pallas_call(
        body,
        grid=(E // blk,),
        in_specs=[
            pl.BlockSpec((blk, FE), lambda i: (i, 0)),
            pl.BlockSpec((FE, HE), lambda i: (0, 0)),
            pl.BlockSpec((1, HE), lambda i: (0, 0)),
            pl.BlockSpec((HE, H), lambda i: (0, 0)),
            pl.BlockSpec((1, H), lambda i: (0, 0)),
        ],
        out_specs=pl.BlockSpec((blk, H), lambda i: (i, 0)),
        out_shape=jax.ShapeDtypeStruct((E, H), jnp.float32),
    )(ef, We, be, Wm2, bm)


def _tc_msg(g, c, Wm1):
    def body(g_ref, c_ref, w_ref, out_ref):
        out_ref[...] = _relu(g_ref[...] @ w_ref[...] + c_ref[...])

    blk = 2048
    return pl.pallas_call(
        body,
        grid=(E // blk,),
        in_specs=[
            pl.BlockSpec((blk, H), lambda i: (i, 0)),
            pl.BlockSpec((blk, H), lambda i: (i, 0)),
            pl.BlockSpec((H, H), lambda i: (0, 0)),
        ],
        out_specs=pl.BlockSpec((blk, H), lambda i: (i, 0)),
        out_shape=jax.ShapeDtypeStruct((E, H), jnp.float32),
    )(g, c, Wm1)


def _tc_update(h, agg2, Wu1, Wu2, bu):
    def body(h_ref, a_ref, w1_ref, w2_ref, b_ref, out_ref):
        a = a_ref[0] + a_ref[1]
        out_ref[...] = _relu(h_ref[...] @ w1_ref[...] + a @ w2_ref[...] + b_ref[...])

    blk = 1024
    return pl.pallas_call(
        body,
        grid=(N // blk,),
        in_specs=[
            pl.BlockSpec((blk, H), lambda i: (i, 0)),
            pl.BlockSpec((NC, blk, H), lambda i: (0, i, 0)),
            pl.BlockSpec((H, H), lambda i: (0, 0)),
            pl.BlockSpec((H, H), lambda i: (0, 0)),
            pl.BlockSpec((1, H), lambda i: (0, 0)),
        ],
        out_specs=pl.BlockSpec((blk, H), lambda i: (i, 0)),
        out_shape=jax.ShapeDtypeStruct((N, H), jnp.float32),
    )(h, agg2, Wu1, Wu2, bu)


def _tc_mlp(h, nm, bi_col, bi_row, W1h, W1m, W2, W3, W4r, b1, b2, b3, b4):
    blk = 1024

    def body(h_ref, nm_ref, bic_ref, bir_ref, w1h_ref, w1m_ref, w2_ref, w3_ref,
             w4_ref, b1_ref, b2_ref, b3_ref, b4_ref, wgt_ref, dd_ref):
        i = pl.program_id(0)
        hb = h_ref[...]
        x = _relu(hb @ w1h_ref[...] + nm_ref[...] @ w1m_ref[...] + b1_ref[...])
        x = _relu(x @ w2_ref[...] + b2_ref[...])
        x = _relu(x @ w3_ref[...] + b3_ref[...])
        logit = jnp.sum(x * w4_ref[...], axis=1, keepdims=True) + b4_ref[...]
        p = jax.nn.sigmoid(logit)
        wgt_ref[...] = hb * p
        t = bic_ref[...]
        acc = jnp.zeros((blk, 1), jnp.int32)
        for kk in range(N // blk):
            ch = bir_ref[:, pl.ds(kk * blk, blk)]
            acc = acc + jnp.sum((ch < t).astype(jnp.int32), axis=1, keepdims=True)
        r = i * blk + lax.broadcasted_iota(jnp.int32, (blk, 1), 0)
        pos = r - acc
        base = t * MAX_ATOMS + pos
        valid = pos < MAX_ATOMS
        tile = lax.shift_right_logical(r, 9)
        half = lax.shift_right_logical(t, 9)
        d0 = jnp.where(valid & (half == 0), base, OUT_ROWS + tile)
        d1 = jnp.where(valid & (half == 1), base, OUT_ROWS + NS + tile)
        dd_ref[...] = jnp.concatenate([d0, d1], axis=1)

    return pl.pallas_call(
        body,
        grid=(N // blk,),
        in_specs=[
            pl.BlockSpec((blk, H), lambda i: (i, 0)),
            pl.BlockSpec((blk, MR), lambda i: (i, 0)),
            pl.BlockSpec((blk, 1), lambda i: (i, 0)),
            pl.BlockSpec((1, N), lambda i: (0, 0)),
            pl.BlockSpec((H, 256), lambda i: (0, 0)),
            pl.BlockSpec((MR, 256), lambda i: (0, 0)),
            pl.BlockSpec((256, 128), lambda i: (0, 0)),
            pl.BlockSpec((128, 64), lambda i: (0, 0)),
            pl.BlockSpec((1, 64), lambda i: (0, 0)),
            pl.BlockSpec((1, 256), lambda i: (0, 0)),
            pl.BlockSpec((1, 128), lambda i: (0, 0)),
            pl.BlockSpec((1, 64), lambda i: (0, 0)),
            pl.BlockSpec((1, 1), lambda i: (0, 0)),
        ],
        out_specs=[
            pl.BlockSpec((blk, H), lambda i: (i, 0)),
            pl.BlockSpec((blk, NC), lambda i: (i, 0)),
        ],
        out_shape=[
            jax.ShapeDtypeStruct((N, H), jnp.float32),
            jax.ShapeDtypeStruct((N, NC), jnp.int32),
        ],
    )(h, nm, bi_col, bi_row, W1h, W1m, W2, W3, W4r, b1, b2, b3, b4)


@jax.jit
def kernel(mol_reprs, node_features, edge_features, edges, batch_indices,
           Wn, bn, We, be, Wm, bm, Wu, bu, W1, b1, W2, b2, W3, b3, W4, b4):
    src = edges[0].astype(jnp.int32).reshape(E // 128, 128)
    dst = edges[1].astype(jnp.int32).reshape(E // 128, 128)
    bi = batch_indices.astype(jnp.int32)
    bi2d = bi.reshape(N // 128, 128)

    Wm1, Wm2 = Wm[:H], Wm[H:]
    Wu1, Wu2 = Wu[:H], Wu[H:]
    W1h, W1m = W1[:H], W1[H:]

    h = _tc_init_h(node_features, Wn, bn.reshape(1, H))
    c = _tc_init_c(edge_features, We, be.reshape(1, HE), Wm2, bm.reshape(1, H))

    for _ in range(NUM_STEPS):
        g = _sc_gather(h, src, H)
        msg = _tc_msg(g, c, Wm1)
        agg2 = _sc_segment_sum(msg, dst)
        h = _tc_update(h, agg2, Wu1, Wu2, bu.reshape(1, H))

    nm = _sc_gather(mol_reprs, bi2d, MR)
    wgt, dd = _tc_mlp(
        h, nm, bi.reshape(N, 1), bi.reshape(1, N),
        W1h, W1m, W2, W3, W4.reshape(1, 64),
        b1.reshape(1, 256), b2.reshape(1, 128), b3.reshape(1, 64),
        b4.reshape(1, 1),
    )
    d2 = jnp.transpose(dd).reshape(NC, N // 128, 128)
    out_pad = _sc_final_scatter(wgt, d2)
    return out_pad[:OUT_ROWS].reshape(B, MAX_ATOMS, H)


# speedup vs baseline: 2.8835x; 1.0581x over previous
import functools

import jax
import jax.numpy as jnp
from jax import lax
from jax.experimental import pallas as pl
from jax.experimental.pallas import tpu as pltpu
from jax.experimental.pallas import tpu_sc as plsc

B = 1024
N = 8192
E = 16384
MR = 256
FN = 64
FE = 16
H = 128
HE = 64
MAX_ATOMS = 24
NUM_STEPS = 8

NC = 2
NS = 16
NW = NC * NS

OUT_ROWS = B * MAX_ATOMS
OUT_PAD = OUT_ROWS + NW
HALF = OUT_ROWS // NC


def _mesh():
    return plsc.VectorSubcoreMesh(
        core_axis_name="c", subcore_axis_name="s", num_cores=NC, num_subcores=NS
    )


def _zero_vmem(ref, rows, cols):
    z = jnp.zeros((16,), jnp.float32)
    cpr = cols // 16

    def body(i, _):
        r = i // cpr
        c = (i % cpr) * 16
        ref[r, pl.ds(c, 16)] = z
        return 0

    lax.fori_loop(0, rows * cpr, body, 0)


def _sc_gather(table, idx2d, d):
    n_chunks = idx2d.shape[0]
    rows = n_chunks * 128
    cpw = n_chunks // NW
    rpw = rows // NW

    @functools.partial(
        pl.kernel,
        out_type=jax.ShapeDtypeStruct((rows, d), jnp.float32),
        mesh=_mesh(),
        scratch_types=[
            pltpu.VMEM((cpw, 128), jnp.int32),
            pltpu.VMEM((rpw, d), jnp.float32),
            pltpu.SemaphoreType.DMA,
            pltpu.SemaphoreType.DMA,
        ],
    )
    def k(table_hbm, idx_hbm, out_hbm, idx_v, rows_v, sem, wsem):
        wid = lax.axis_index("s") * NC + lax.axis_index("c")
        pltpu.sync_copy(idx_hbm.at[pl.ds(wid * cpw, cpw)], idx_v)
        descs = [
            pltpu.async_copy(
                table_hbm.at[idx_v.at[j]], rows_v.at[pl.ds(j * 128, 128)], sem
            )
            for j in range(cpw)
        ]
        wdescs = []
        for j in range(cpw):
            descs[j].wait()
            wdescs.append(
                pltpu.async_copy(
                    rows_v.at[pl.ds(j * 128, 128)],
                    out_hbm.at[pl.ds(wid * rpw + j * 128, 128)],
                    wsem,
                )
            )
        for dsc in wdescs:
            dsc.wait()

    return k(table, idx2d)


def _sc_segment_sum(msg, dst2d):
    epw = E // NW
    cpw = epw // 128
    npt = N // NS

    @functools.partial(
        pl.kernel,
        out_type=jax.ShapeDtypeStruct((NC, N, H), jnp.float32),
        mesh=_mesh(),
        scratch_types=[
            pltpu.VMEM((cpw, 128), jnp.int32),
            pltpu.VMEM((2, 128, H), jnp.float32),
            pltpu.VMEM((128, H), jnp.float32),
            pltpu.VMEM_SHARED((N, H), jnp.float32),
            pltpu.SemaphoreType.DMA,
            pltpu.SemaphoreType.DMA,
        ],
    )
    def k(msg_hbm, dst_hbm, out_hbm, idx_v, rows_v, zbuf, acc, sem, lsem):
        cid = lax.axis_index("c")
        sid = lax.axis_index("s")
        wid = sid * NC + cid
        d_idx = pltpu.async_copy(dst_hbm.at[pl.ds(wid * cpw, cpw)], idx_v, sem)
        loads = [
            pltpu.async_copy(
                msg_hbm.at[pl.ds(wid * epw + j * 128, 128)], rows_v.at[j % 2], lsem
            )
            for j in range(2)
        ]
        _zero_vmem(zbuf, 128, H)
        for q in range(npt // 128):
            pltpu.sync_copy(zbuf, acc.at[pl.ds(sid * npt + q * 128, 128)])
        plsc.subcore_barrier()
        d_idx.wait()
        for j in range(cpw):
            loads[j].wait()
            pltpu.sync_copy(rows_v.at[j % 2], acc.at[idx_v.at[j]], add=True)
            if j + 2 < cpw:
                loads.append(
                    pltpu.async_copy(
                        msg_hbm.at[pl.ds(wid * epw + (j + 2) * 128, 128)],
                        rows_v.at[j % 2],
                        lsem,
                    )
                )
        plsc.subcore_barrier()
        pltpu.sync_copy(
            acc.at[pl.ds(sid * npt, npt)], out_hbm.at[cid, pl.ds(sid * npt, npt)]
        )

    return k(msg, dst2d)


def _sc_final_scatter(wgt, d2):
    rpw = N // NS
    cpw = rpw // 128
    zrows = HALF // NS

    @functools.partial(
        pl.kernel,
        out_type=jax.ShapeDtypeStruct((OUT_PAD, H), jnp.float32),
        mesh=_mesh(),
        scratch_types=[
            pltpu.VMEM((cpw, 128), jnp.int32),
            pltpu.VMEM((rpw, H), jnp.float32),
            pltpu.VMEM((256, H), jnp.float32),
            pltpu.SemaphoreType.DMA,
        ],
    )
    def k(wgt_hbm, d_hbm, out_hbm, idx_v, rows_v, zbuf, sem):
        cid = lax.axis_index("c")
        sid = lax.axis_index("s")
        d_idx = pltpu.async_copy(d_hbm.at[cid, pl.ds(sid * cpw, cpw)], idx_v, sem)
        d_rows = pltpu.async_copy(wgt_hbm.at[pl.ds(sid * rpw, rpw)], rows_v, sem)
        _zero_vmem(zbuf, 256, H)
        for q in range(zrows // 256):
            pltpu.sync_copy(
                zbuf, out_hbm.at[pl.ds(cid * HALF + sid * zrows + q * 256, 256)]
            )
        plsc.subcore_barrier()
        d_idx.wait()
        d_rows.wait()
        for j in range(cpw):
            pltpu.sync_copy(rows_v.at[pl.ds(j * 128, 128)], out_hbm.at[idx_v.at[j]])

    return k(wgt, d2)


def _relu(x):
    return jnp.maximum(x, 0.0)


def _tc_init_h(nf, Wn, bn):
    def body(nf_ref, wn_ref, bn_ref, out_ref):
        out_ref[...] = _relu(nf_ref[...] @ wn_ref[...] + bn_ref[...])

    blk = 1024
    return pl.pallas_call(
        body,
        grid=(N // blk,),
        in_specs=[
            pl.BlockSpec((blk, FN), lambda i: (i, 0)),
            pl.BlockSpec((FN, H), lambda i: (0, 0)),
            pl.BlockSpec((1, H), lambda i: (0, 0)),
        ],
        out_specs=pl.BlockSpec((blk, H), lambda i: (i, 0)),
        out_shape=jax.ShapeDtypeStruct((N, H), jnp.float32),
    )(nf, Wn, bn)


def _tc_init_c(ef, We, be, Wm2, bm):
    def body(ef_ref, we_ref, be_ref, wm2_ref, bm_ref, out_ref):
        eh = _relu(ef_ref[...] @ we_ref[...] + be_ref[...])
        out_ref[...] = eh @ wm2_ref[...] + bm_ref[...]

    blk = 2048
    return pl.pallas_call(
        body,
        grid=(E // blk,),
        in_specs=[
            pl.BlockSpec((blk, FE), lambda i: (i, 0)),
            pl.BlockSpec((FE, HE), lambda i: (0, 0)),
            pl.BlockSpec((1, HE), lambda i: (0, 0)),
            pl.BlockSpec((HE, H), lambda i: (0, 0)),
            pl.BlockSpec((1, H), lambda i: (0, 0)),
        ],
        out_specs=pl.BlockSpec((blk, H), lambda i: (i, 0)),
        out_shape=jax.ShapeDtypeStruct((E, H), jnp.float32),
    )(ef, We, be, Wm2, bm)


def _tc_msg(g, c, Wm1):
    def body(g_ref, c_ref, w_ref, out_ref):
        out_ref[...] = _relu(g_ref[...] @ w_ref[...] + c_ref[...])

    blk = 2048
    return pl.pallas_call(
        body,
        grid=(E // blk,),
        in_specs=[
            pl.BlockSpec((blk, H), lambda i: (i, 0)),
            pl.BlockSpec((blk, H), lambda i: (i, 0)),
            pl.BlockSpec((H, H), lambda i: (0, 0)),
        ],
        out_specs=pl.BlockSpec((blk, H), lambda i: (i, 0)),
        out_shape=jax.ShapeDtypeStruct((E, H), jnp.float32),
    )(g, c, Wm1)


def _tc_update(h, agg2, Wu1, Wu2, bu):
    def body(h_ref, a_ref, w1_ref, w2_ref, b_ref, out_ref):
        a = a_ref[0] + a_ref[1]
        out_ref[...] = _relu(h_ref[...] @ w1_ref[...] + a @ w2_ref[...] + b_ref[...])

    blk = 1024
    return pl.pallas_call(
        body,
        grid=(N // blk,),
        in_specs=[
            pl.BlockSpec((blk, H), lambda i: (i, 0)),
            pl.BlockSpec((NC, blk, H), lambda i: (0, i, 0)),
            pl.BlockSpec((H, H), lambda i: (0, 0)),
            pl.BlockSpec((H, H), lambda i: (0, 0)),
            pl.BlockSpec((1, H), lambda i: (0, 0)),
        ],
        out_specs=pl.BlockSpec((blk, H), lambda i: (i, 0)),
        out_shape=jax.ShapeDtypeStruct((N, H), jnp.float32),
    )(h, agg2, Wu1, Wu2, bu)


def _tc_mlp(h, nm, bi_col, bi_row, W1h, W1m, W2, W3, W4r, b1, b2, b3, b4):
    blk = 1024

    def body(h_ref, nm_ref, bic_ref, bir_ref, w1h_ref, w1m_ref, w2_ref, w3_ref,
             w4_ref, b1_ref, b2_ref, b3_ref, b4_ref, wgt_ref, dd_ref):
        i = pl.program_id(0)
        hb = h_ref[...]
        x = _relu(hb @ w1h_ref[...] + nm_ref[...] @ w1m_ref[...] + b1_ref[...])
        x = _relu(x @ w2_ref[...] + b2_ref[...])
        x = _relu(x @ w3_ref[...] + b3_ref[...])
        logit = jnp.sum(x * w4_ref[...], axis=1, keepdims=True) + b4_ref[...]
        p = jax.nn.sigmoid(logit)
        wgt_ref[...] = hb * p
        t = bic_ref[...]
        acc = jnp.zeros((blk, 1), jnp.int32)
        for kk in range(N // blk):
            ch = bir_ref[:, pl.ds(kk * blk, blk)]
            acc = acc + jnp.sum((ch < t).astype(jnp.int32), axis=1, keepdims=True)
        r = i * blk + lax.broadcasted_iota(jnp.int32, (blk, 1), 0)
        pos = r - acc
        base = t * MAX_ATOMS + pos
        valid = pos < MAX_ATOMS
        tile = lax.shift_right_logical(r, 9)
        half = lax.shift_right_logical(t, 9)
        d0 = jnp.where(valid & (half == 0), base, OUT_ROWS + tile)
        d1 = jnp.where(valid & (half == 1), base, OUT_ROWS + NS + tile)
        dd_ref[...] = jnp.concatenate([d0, d1], axis=1)

    return pl.pallas_call(
        body,
        grid=(N // blk,),
        in_specs=[
            pl.BlockSpec((blk, H), lambda i: (i, 0)),
            pl.BlockSpec((blk, MR), lambda i: (i, 0)),
            pl.BlockSpec((blk, 1), lambda i: (i, 0)),
            pl.BlockSpec((1, N), lambda i: (0, 0)),
            pl.BlockSpec((H, 256), lambda i: (0, 0)),
            pl.BlockSpec((MR, 256), lambda i: (0, 0)),
            pl.BlockSpec((256, 128), lambda i: (0, 0)),
            pl.BlockSpec((128, 64), lambda i: (0, 0)),
            pl.BlockSpec((1, 64), lambda i: (0, 0)),
            pl.BlockSpec((1, 256), lambda i: (0, 0)),
            pl.BlockSpec((1, 128), lambda i: (0, 0)),
            pl.BlockSpec((1, 64), lambda i: (0, 0)),
            pl.BlockSpec((1, 1), lambda i: (0, 0)),
        ],
        out_specs=[
            pl.BlockSpec((blk, H), lambda i: (i, 0)),
            pl.BlockSpec((blk, NC), lambda i: (i, 0)),
        ],
        out_shape=[
            jax.ShapeDtypeStruct((N, H), jnp.float32),
            jax.ShapeDtypeStruct((N, NC), jnp.int32),
        ],
    )(h, nm, bi_col, bi_row, W1h, W1m, W2, W3, W4r, b1, b2, b3, b4)


@jax.jit
def kernel(mol_reprs, node_features, edge_features, edges, batch_indices,
           Wn, bn, We, be, Wm, bm, Wu, bu, W1, b1, W2, b2, W3, b3, W4, b4):
    src = edges[0].astype(jnp.int32).reshape(E // 128, 128)
    dst = edges[1].astype(jnp.int32).reshape(E // 128, 128)
    bi = batch_indices.astype(jnp.int32)
    bi2d = bi.reshape(N // 128, 128)

    Wm1, Wm2 = Wm[:H], Wm[H:]
    Wu1, Wu2 = Wu[:H], Wu[H:]
    W1h, W1m = W1[:H], W1[H:]

    h = _tc_init_h(node_features, Wn, bn.reshape(1, H))
    c = _tc_init_c(edge_features, We, be.reshape(1, HE), Wm2, bm.reshape(1, H))

    for _ in range(NUM_STEPS):
        g = _sc_gather(h, src, H)
        msg = _tc_msg(g, c, Wm1)
        agg2 = _sc_segment_sum(msg, dst)
        h = _tc_update(h, agg2, Wu1, Wu2, bu.reshape(1, H))

    nm = _sc_gather(mol_reprs, bi2d, MR)
    wgt, dd = _tc_mlp(
        h, nm, bi.reshape(N, 1), bi.reshape(1, N),
        W1h, W1m, W2, W3, W4.reshape(1, 64),
        b1.reshape(1, 256), b2.reshape(1, 128), b3.reshape(1, 64),
        b4.reshape(1, 1),
    )
    d2 = jnp.transpose(dd).reshape(NC, N // 128, 128)
    out_pad = _sc_final_scatter(wgt, d2)
    return out_pad[:OUT_ROWS].reshape(B, MAX_ATOMS, H)
